# Initial kernel scaffold; baseline (speedup 1.0000x reference)
import jax
import jax.numpy as jnp
from jax.experimental import pallas as pl


def kernel(x, x_lengths, emb_weight):
    raise NotImplementedError("write your pallas kernel here")


# speedup vs baseline: 3.2663x; 3.2663x over previous
import functools
import math

import jax
import jax.numpy as jnp
from jax import lax
from jax.experimental import pallas as pl
from jax.experimental.pallas import tpu as pltpu
from jax.experimental.pallas import tpu_sc as plsc

N_VOCAB = 100000
HIDDEN = 64
BATCH = 4096
SEQ = 200
SCALE = math.sqrt(HIDDEN)

NW = 32
TOTAL = BATCH * SEQ
PER_W = TOTAL // NW
CHUNK = 128
NCHUNK = PER_W // CHUNK


def _scale_body(t_ref, o_ref):
    o_ref[...] = t_ref[...] * SCALE


def _scale_table(emb_weight):
    t2 = emb_weight.reshape(N_VOCAB // 2, 2 * HIDDEN)
    scaled = pl.pallas_call(
        _scale_body,
        grid=(125,),
        in_specs=[pl.BlockSpec((400, 128), lambda i: (i, 0))],
        out_specs=pl.BlockSpec((400, 128), lambda i: (i, 0)),
        out_shape=jax.ShapeDtypeStruct((N_VOCAB // 2, 2 * HIDDEN), jnp.float32),
    )(t2)
    return scaled.reshape(N_VOCAB, HIDDEN)


def _gather_kernel(x_hbm, table_hbm, out_hbm, idx_v, rows_v, sem):
    c = lax.axis_index("c")
    s = lax.axis_index("s")
    wid = s * 2 + c
    pltpu.sync_copy(x_hbm.at[wid], idx_v)

    def step(j, carry):
        pltpu.async_copy(table_hbm.at[idx_v.at[j]], rows_v, sem).wait()
        pltpu.sync_copy(rows_v, out_hbm.at[pl.ds(wid * PER_W + j * CHUNK, CHUNK)])
        return carry

    lax.fori_loop(0, NCHUNK, step, 0)


@jax.jit
def kernel(x, x_lengths, emb_weight):
    del x_lengths
    table = _scale_table(emb_weight)
    xw = x.reshape(NW, NCHUNK, CHUNK).astype(jnp.int32)

    mesh = plsc.VectorSubcoreMesh(core_axis_name="c", subcore_axis_name="s")
    gather = functools.partial(
        pl.kernel,
        mesh=mesh,
        out_type=jax.ShapeDtypeStruct((TOTAL, HIDDEN), jnp.float32),
        scratch_types=[
            pltpu.VMEM((NCHUNK, CHUNK), jnp.int32),
            pltpu.VMEM((CHUNK, HIDDEN), jnp.float32),
            pltpu.SemaphoreType.DMA,
        ],
        compiler_params=pltpu.CompilerParams(use_tc_tiling_on_sc=False),
    )(_gather_kernel)
    out = gather(xw, table)
    return out.reshape(BATCH, SEQ, HIDDEN)


# speedup vs baseline: 4.0018x; 1.2252x over previous
import functools
import math

import jax
import jax.numpy as jnp
from jax import lax
from jax.experimental import pallas as pl
from jax.experimental.pallas import tpu as pltpu
from jax.experimental.pallas import tpu_sc as plsc

N_VOCAB = 100000
HIDDEN = 64
BATCH = 4096
SEQ = 200
SCALE = math.sqrt(HIDDEN)

NW = 32
NBG = BATCH // 128
NUNIT = SEQ * NBG
PER_W = NUNIT // NW
TBLK = 2048


def _scale_body(t_ref, o_ref):
    o_ref[...] = t_ref[...].T * SCALE


def _scale_table(emb_weight):
    embT = emb_weight.T
    return pl.pallas_call(
        _scale_body,
        grid=((N_VOCAB + TBLK - 1) // TBLK,),
        in_specs=[pl.BlockSpec((HIDDEN, TBLK), lambda i: (0, i))],
        out_specs=pl.BlockSpec((TBLK, HIDDEN), lambda i: (i, 0)),
        out_shape=jax.ShapeDtypeStruct((N_VOCAB, HIDDEN), jnp.float32),
    )(embT)


def _gather_kernel(
    x_hbm, table_hbm, out_hbm,
    idx_v, rows0, rows1, tr0, tr1, gs0, gs1, os0, os1,
):
    c = lax.axis_index("c")
    s_ax = lax.axis_index("s")
    wid = s_ax * 2 + c
    pltpu.sync_copy(x_hbm.at[wid], idx_v)
    lane = lax.iota(jnp.int32, 16)
    base_u = wid * PER_W

    def gather_start(t, rbuf, sem):
        pltpu.make_async_copy(table_hbm.at[idx_v.at[t]], rbuf, sem).start()

    def gather_wait(rbuf, sem):
        pltpu.make_async_copy(table_hbm.at[idx_v.at[0]], rbuf, sem).wait()

    def transpose(rbuf, tbuf):
        @plsc.parallel_loop(0, HIDDEN, unroll=4)
        def h_body(h):
            hg = h // 8
            hi = h % 8
            hvec = jnp.full((16,), h, jnp.int32)
            for g in range(8):
                v = plsc.load_gather(rbuf, [lane + (g * 16), hvec])
                tbuf[hg, hi, pl.ds(g * 16, 16)] = v

    def out_ref(t):
        u = base_u + t
        return out_hbm.at[u // NBG, :, u % NBG]

    def out_start(t, tbuf, sem):
        pltpu.make_async_copy(tbuf, out_ref(t), sem).start()

    def out_wait(tbuf, sem):
        pltpu.make_async_copy(tbuf, out_ref(0), sem).wait()

    gather_start(0, rows0, gs0)
    gather_start(1, rows1, gs1)

    def body(t2, carry):
        t0 = 2 * t2
        gather_wait(rows0, gs0)

        @pl.when(t2 > 0)
        def _():
            out_wait(tr0, os0)

        transpose(rows0, tr0)

        @pl.when(t2 < PER_W // 2 - 1)
        def _():
            gather_start(t0 + 2, rows0, gs0)

        out_start(t0, tr0, os0)

        gather_wait(rows1, gs1)

        @pl.when(t2 > 0)
        def _():
            out_wait(tr1, os1)

        transpose(rows1, tr1)

        @pl.when(t2 < PER_W // 2 - 1)
        def _():
            gather_start(t0 + 3, rows1, gs1)

        out_start(t0 + 1, tr1, os1)
        return carry

    lax.fori_loop(0, PER_W // 2, body, 0)
    out_wait(tr0, os0)
    out_wait(tr1, os1)


@jax.jit
def kernel(x, x_lengths, emb_weight):
    del x_lengths
    table = _scale_table(emb_weight)
    xw = x.T.astype(jnp.int32).reshape(NW, PER_W, 128)

    mesh = plsc.VectorSubcoreMesh(core_axis_name="c", subcore_axis_name="s")
    gather = functools.partial(
        pl.kernel,
        mesh=mesh,
        out_type=jax.ShapeDtypeStruct((SEQ, 8, NBG, 8, 128), jnp.float32),
        scratch_types=[
            pltpu.VMEM((PER_W, 128), jnp.int32),
            pltpu.VMEM((128, HIDDEN), jnp.float32),
            pltpu.VMEM((128, HIDDEN), jnp.float32),
            pltpu.VMEM((8, 8, 128), jnp.float32),
            pltpu.VMEM((8, 8, 128), jnp.float32),
            pltpu.SemaphoreType.DMA,
            pltpu.SemaphoreType.DMA,
            pltpu.SemaphoreType.DMA,
            pltpu.SemaphoreType.DMA,
        ],
        compiler_params=pltpu.CompilerParams(
            use_tc_tiling_on_sc=False, needs_layout_passes=False
        ),
    )(_gather_kernel)
    out5 = gather(xw, table)
    return out5.transpose((2, 4, 0, 1, 3)).reshape(BATCH, SEQ, HIDDEN)


# speedup vs baseline: 4.2262x; 1.0561x over previous
import functools
import math

import jax
import jax.numpy as jnp
from jax import lax
from jax.experimental import pallas as pl
from jax.experimental.pallas import tpu as pltpu
from jax.experimental.pallas import tpu_sc as plsc

N_VOCAB = 100000
HIDDEN = 64
BATCH = 4096
SEQ = 200
SCALE = math.sqrt(HIDDEN)

NW = 32
NBG = BATCH // 128
NUNIT = SEQ * NBG
PER_W = NUNIT // NW
TBLK = 2048


def _scale_body(t_ref, o_ref):
    o_ref[:, 0:HIDDEN] = t_ref[...].T * SCALE


def _scale_table(emb_weight):
    embT = emb_weight.T
    scaled = pl.pallas_call(
        _scale_body,
        grid=((N_VOCAB + TBLK - 1) // TBLK,),
        in_specs=[pl.BlockSpec((HIDDEN, TBLK), lambda i: (0, i))],
        out_specs=pl.BlockSpec((TBLK, 2 * HIDDEN), lambda i: (i, 0)),
        out_shape=jax.ShapeDtypeStruct((N_VOCAB, 2 * HIDDEN), jnp.float32),
    )(embT)
    return scaled.reshape(2 * N_VOCAB, HIDDEN)


def _gather_kernel(
    x_hbm, table_hbm, out_hbm,
    idx_v, rows0, rows1, tr0, tr1, gs0, gs1, os0, os1,
):
    c = lax.axis_index("c")
    s_ax = lax.axis_index("s")
    wid = s_ax * 2 + c
    pltpu.sync_copy(x_hbm.at[wid], idx_v)
    lane = lax.iota(jnp.int32, 16)
    base_u = wid * PER_W

    def gather_start(t, rbuf, sem):
        pltpu.make_async_copy(table_hbm.at[idx_v.at[t]], rbuf, sem).start()

    def gather_wait(rbuf, sem):
        pltpu.make_async_copy(table_hbm.at[idx_v.at[0]], rbuf, sem).wait()

    def transpose(rbuf, tbuf):
        @plsc.parallel_loop(0, HIDDEN, unroll=4)
        def h_body(h):
            hg = h // 8
            hi = h % 8
            hvec = jnp.full((16,), h, jnp.int32)
            for g in range(8):
                v = plsc.load_gather(rbuf, [lane + (g * 16), hvec])
                tbuf[hg, hi, pl.ds(g * 16, 16)] = v

    def out_ref(t):
        u = base_u + t
        return out_hbm.at[u // NBG, :, u % NBG]

    def out_start(t, tbuf, sem):
        pltpu.make_async_copy(tbuf, out_ref(t), sem).start()

    def out_wait(tbuf, sem):
        pltpu.make_async_copy(tbuf, out_ref(0), sem).wait()

    gather_start(0, rows0, gs0)
    gather_start(1, rows1, gs1)

    def body(t2, carry):
        t0 = 2 * t2
        gather_wait(rows0, gs0)

        @pl.when(t2 > 0)
        def _():
            out_wait(tr0, os0)

        transpose(rows0, tr0)

        @pl.when(t2 < PER_W // 2 - 1)
        def _():
            gather_start(t0 + 2, rows0, gs0)

        out_start(t0, tr0, os0)

        gather_wait(rows1, gs1)

        @pl.when(t2 > 0)
        def _():
            out_wait(tr1, os1)

        transpose(rows1, tr1)

        @pl.when(t2 < PER_W // 2 - 1)
        def _():
            gather_start(t0 + 3, rows1, gs1)

        out_start(t0 + 1, tr1, os1)
        return carry

    lax.fori_loop(0, PER_W // 2, body, 0)
    out_wait(tr0, os0)
    out_wait(tr1, os1)


@jax.jit
def kernel(x, x_lengths, emb_weight):
    del x_lengths
    table = _scale_table(emb_weight)
    xw = (x.T.astype(jnp.int32) * 2).reshape(NW, PER_W, 128)

    mesh = plsc.VectorSubcoreMesh(core_axis_name="c", subcore_axis_name="s")
    gather = functools.partial(
        pl.kernel,
        mesh=mesh,
        out_type=jax.ShapeDtypeStruct((SEQ, 8, NBG, 8, 128), jnp.float32),
        scratch_types=[
            pltpu.VMEM((PER_W, 128), jnp.int32),
            pltpu.VMEM((128, HIDDEN), jnp.float32),
            pltpu.VMEM((128, HIDDEN), jnp.float32),
            pltpu.VMEM((8, 8, 128), jnp.float32),
            pltpu.VMEM((8, 8, 128), jnp.float32),
            pltpu.SemaphoreType.DMA,
            pltpu.SemaphoreType.DMA,
            pltpu.SemaphoreType.DMA,
            pltpu.SemaphoreType.DMA,
        ],
        compiler_params=pltpu.CompilerParams(
            use_tc_tiling_on_sc=False, needs_layout_passes=False
        ),
    )(_gather_kernel)
    out5 = gather(xw, table)
    return out5.transpose((2, 4, 0, 1, 3)).reshape(BATCH, SEQ, HIDDEN)
